# baseline (speedup 1.0000x reference)
import functools

import jax
import jax.numpy as jnp
from jax import lax
from jax.experimental import pallas as pl
from jax.experimental.pallas import tpu as pltpu
from jax.experimental.pallas import tpu_sc as plsc

_DIM = 33
_TBL = _DIM ** 3
_BINSIZE = 1.000001 / (_DIM - 1)
_NW = 32
_NPIX = 8 * 512 * 512
_PER_W = _NPIX // _NW
_IMG = 512 * 512
_CHUNK = 4096
_L = 16


@functools.partial(
    pl.kernel,
    out_type=jax.ShapeDtypeStruct((24, _IMG), jnp.float32),
    mesh=plsc.VectorSubcoreMesh(core_axis_name="c", subcore_axis_name="s"),
    compiler_params=pltpu.CompilerParams(needs_layout_passes=False),
    scratch_types=[
        pltpu.VMEM((3 * _TBL,), jnp.float32),
        pltpu.VMEM((_CHUNK,), jnp.float32),
        pltpu.VMEM((_CHUNK,), jnp.float32),
        pltpu.VMEM((_CHUNK,), jnp.float32),
    ],
)
def _lut3d_sc(lut_hbm, x_hbm, out_hbm, lut_v, rv, gv, bv):
    wid = lax.axis_index("s") * 2 + lax.axis_index("c")
    img = wid // 4
    base_px = (wid % 4) * _PER_W

    pltpu.sync_copy(lut_hbm, lut_v)

    def chunk_body(ci, carry):
        off = base_px + ci * _CHUNK
        pltpu.sync_copy(x_hbm.at[3 * img + 0, pl.ds(off, _CHUNK)], rv)
        pltpu.sync_copy(x_hbm.at[3 * img + 1, pl.ds(off, _CHUNK)], gv)
        pltpu.sync_copy(x_hbm.at[3 * img + 2, pl.ds(off, _CHUNK)], bv)

        inv = 1.0 / _BINSIZE

        @plsc.parallel_loop(0, _CHUNK, _L, unroll=4)
        def vec_body(s):
            r = rv[pl.ds(s, _L)]
            g = gv[pl.ds(s, _L)]
            b = bv[pl.ds(s, _L)]
            rs = r * inv
            gs = g * inv
            bs = b * inv
            ri = rs.astype(jnp.int32)
            gi = gs.astype(jnp.int32)
            bi = bs.astype(jnp.int32)
            rd = rs - ri.astype(jnp.float32)
            gd = gs - gi.astype(jnp.float32)
            bd = bs - bi.astype(jnp.float32)
            base = ri + gi * _DIM + bi * (_DIM * _DIM)
            outs = []
            for c in range(3):
                bc = base + c * _TBL
                v000 = plsc.load_gather(lut_v, [bc])
                v100 = plsc.load_gather(lut_v, [bc + 1])
                v010 = plsc.load_gather(lut_v, [bc + _DIM])
                v110 = plsc.load_gather(lut_v, [bc + (_DIM + 1)])
                v001 = plsc.load_gather(lut_v, [bc + _DIM * _DIM])
                v101 = plsc.load_gather(lut_v, [bc + (_DIM * _DIM + 1)])
                v011 = plsc.load_gather(lut_v, [bc + (_DIM * _DIM + _DIM)])
                v111 = plsc.load_gather(lut_v, [bc + (_DIM * _DIM + _DIM + 1)])
                v00 = v000 + rd * (v100 - v000)
                v10 = v010 + rd * (v110 - v010)
                v01 = v001 + rd * (v101 - v001)
                v11 = v011 + rd * (v111 - v011)
                v0 = v00 + gd * (v10 - v00)
                v1 = v01 + gd * (v11 - v01)
                outs.append(v0 + bd * (v1 - v0))
            rv[pl.ds(s, _L)] = outs[0]
            gv[pl.ds(s, _L)] = outs[1]
            bv[pl.ds(s, _L)] = outs[2]

        pltpu.sync_copy(rv, out_hbm.at[3 * img + 0, pl.ds(off, _CHUNK)])
        pltpu.sync_copy(gv, out_hbm.at[3 * img + 1, pl.ds(off, _CHUNK)])
        pltpu.sync_copy(bv, out_hbm.at[3 * img + 2, pl.ds(off, _CHUNK)])
        return carry

    lax.fori_loop(0, _PER_W // _CHUNK, chunk_body, 0)


def kernel(lut, x):
    lut_flat = lut.reshape(3 * _TBL)
    x_flat = x.reshape(24, _IMG)
    out = _lut3d_sc(lut_flat, x_flat)
    return out.reshape(8, 3, 512, 512)


# speedup vs baseline: 1.1167x; 1.1167x over previous
import jax
import jax.numpy as jnp
from jax import lax
from jax.experimental import pallas as pl
from jax.experimental.pallas import tpu as pltpu
from jax.experimental.pallas import tpu_sc as plsc

_DIM = 33
_DIM2 = _DIM * _DIM
_TBL = _DIM ** 3
_BINSIZE = 1.000001 / (_DIM - 1)
_NW = 32
_NPIX = 8 * 512 * 512
_PER_W = _NPIX // _NW
_IMG = 512 * 512
_CHUNK = 4096
_L = 16


@pl.kernel(
    out_type=jax.ShapeDtypeStruct((24, _IMG), jnp.float32),
    mesh=plsc.VectorSubcoreMesh(core_axis_name="c", subcore_axis_name="s"),
    compiler_params=pltpu.CompilerParams(needs_layout_passes=False),
    scratch_types=[
        pltpu.VMEM((3 * _TBL,), jnp.int32),
        pltpu.VMEM((_CHUNK,), jnp.float32),
        pltpu.VMEM((_CHUNK,), jnp.float32),
        pltpu.VMEM((_CHUNK,), jnp.float32),
    ],
)
def _lut3d_sc(lut_hbm, x_hbm, out_hbm, lut_v, rv, gv, bv):
    wid = lax.axis_index("s") * 2 + lax.axis_index("c")
    img = wid // 4
    base_px = (wid % 4) * _PER_W

    pltpu.sync_copy(lut_hbm, lut_v)
    inv = 1.0 / _BINSIZE

    def chunk_body(ci, carry):
        off = base_px + ci * _CHUNK
        pltpu.sync_copy(x_hbm.at[3 * img + 0, pl.ds(off, _CHUNK)], rv)
        pltpu.sync_copy(x_hbm.at[3 * img + 1, pl.ds(off, _CHUNK)], gv)
        pltpu.sync_copy(x_hbm.at[3 * img + 2, pl.ds(off, _CHUNK)], bv)

        @plsc.parallel_loop(0, _CHUNK, _L, unroll=4)
        def vec_body(s):
            r = rv[pl.ds(s, _L)]
            g = gv[pl.ds(s, _L)]
            b = bv[pl.ds(s, _L)]
            rs = r * inv
            gs = g * inv
            bs = b * inv
            ri = rs.astype(jnp.int32)
            gi = gs.astype(jnp.int32)
            bi = bs.astype(jnp.int32)
            rd = rs - ri.astype(jnp.float32)
            gd = gs - gi.astype(jnp.float32)
            bd = bs - bi.astype(jnp.float32)
            base = ri + gi * _DIM + bi * _DIM2

            def unpk(w):
                lo = plsc.bitcast(w << 16, jnp.float32)
                hi = plsc.bitcast(w & jnp.int32(-65536), jnp.float32)
                return lo, hi

            outs = []
            for c in range(3):
                bc = base + c * _TBL
                v000, v100 = unpk(plsc.load_gather(lut_v, [bc]))
                v010, v110 = unpk(plsc.load_gather(lut_v, [bc + _DIM]))
                v001, v101 = unpk(plsc.load_gather(lut_v, [bc + _DIM2]))
                v011, v111 = unpk(plsc.load_gather(lut_v, [bc + (_DIM2 + _DIM)]))
                v00 = v000 + rd * (v100 - v000)
                v10 = v010 + rd * (v110 - v010)
                v01 = v001 + rd * (v101 - v001)
                v11 = v011 + rd * (v111 - v011)
                v0 = v00 + gd * (v10 - v00)
                v1 = v01 + gd * (v11 - v01)
                outs.append(v0 + bd * (v1 - v0))
            rv[pl.ds(s, _L)] = outs[0]
            gv[pl.ds(s, _L)] = outs[1]
            bv[pl.ds(s, _L)] = outs[2]

        pltpu.sync_copy(rv, out_hbm.at[3 * img + 0, pl.ds(off, _CHUNK)])
        pltpu.sync_copy(gv, out_hbm.at[3 * img + 1, pl.ds(off, _CHUNK)])
        pltpu.sync_copy(bv, out_hbm.at[3 * img + 2, pl.ds(off, _CHUNK)])
        return carry

    lax.fori_loop(0, _PER_W // _CHUNK, chunk_body, 0)


def _pack_pairs(lut):
    t = lut.reshape(3, _TBL)
    tn = jnp.concatenate([t[:, 1:], t[:, -1:]], axis=1)
    lo = lax.bitcast_convert_type(t.astype(jnp.bfloat16), jnp.uint16)
    hi = lax.bitcast_convert_type(tn.astype(jnp.bfloat16), jnp.uint16)
    packed = lo.astype(jnp.uint32) | (hi.astype(jnp.uint32) << 16)
    return lax.bitcast_convert_type(packed, jnp.int32).reshape(3 * _TBL)


def kernel(lut, x):
    lut_pairs = _pack_pairs(lut)
    x_flat = x.reshape(24, _IMG)
    out = _lut3d_sc(lut_pairs, x_flat)
    return out.reshape(8, 3, 512, 512)


# speedup vs baseline: 2.2907x; 2.0513x over previous
import jax
import jax.numpy as jnp
from jax import lax
from jax.experimental import pallas as pl
from jax.experimental.pallas import tpu as pltpu
from jax.experimental.pallas import tpu_sc as plsc

_DIM = 33
_DIM2 = _DIM * _DIM
_TBL = _DIM ** 3
_BINSIZE = 1.000001 / (_DIM - 1)
_W = 512
_CROWS = 2
_NCHUNK = 128 // _CROWS
_L = 16

_buf = lambda: pltpu.VMEM((_CROWS, _W), jnp.float32)


@pl.kernel(
    out_type=jax.ShapeDtypeStruct((8, 3, 512, 512), jnp.float32),
    mesh=plsc.VectorSubcoreMesh(core_axis_name="c", subcore_axis_name="s"),
    compiler_params=pltpu.CompilerParams(needs_layout_passes=False),
    scratch_types=[pltpu.VMEM((3 * _TBL,), jnp.int32)]
    + [_buf() for _ in range(12)]
    + [pltpu.SemaphoreType.DMA for _ in range(4)],
)
def _lut3d_sc(lut_hbm, x_hbm, out_hbm, lut_v,
              i00, i01, i02, i10, i11, i12,
              o00, o01, o02, o10, o11, o12,
              sin0, sin1, sout0, sout1):
    wid = lax.axis_index("s") * 2 + lax.axis_index("c")
    img = wid // 4
    rows0 = (wid % 4) * 128

    ins = ((i00, i01, i02), (i10, i11, i12))
    outs = ((o00, o01, o02), (o10, o11, o12))
    sins = (sin0, sin1)
    souts = (sout0, sout1)

    def in_copies(ci, p):
        row = rows0 + ci * _CROWS
        return [pltpu.make_async_copy(
            x_hbm.at[img, c, pl.ds(row, _CROWS), :], ins[p][c], sins[p])
            for c in range(3)]

    def out_copies(ci, p):
        row = rows0 + ci * _CROWS
        return [pltpu.make_async_copy(
            outs[p][c], out_hbm.at[img, c, pl.ds(row, _CROWS), :], souts[p])
            for c in range(3)]

    pltpu.sync_copy(lut_hbm, lut_v)
    inv = 1.0 / _BINSIZE

    for cp in in_copies(0, 0):
        cp.start()

    @pl.loop(0, _NCHUNK, step=2)
    def chunk_pair(base):
        for b in range(2):
            ci = base + b
            p, q = b, 1 - b

            @pl.when(ci + 1 < _NCHUNK)
            def _prefetch():
                for cp in in_copies(ci + 1, q):
                    cp.start()

            for cp in in_copies(ci, p):
                cp.wait()

            @pl.when(ci >= 2)
            def _drain():
                for cp in out_copies(ci - 2, p):
                    cp.wait()

            rv, gv, bv = ins[p]
            ov0, ov1, ov2 = outs[p]

            @plsc.parallel_loop(0, _W, _L, unroll=2)
            def vec_body(s):
                for jj in range(_CROWS):
                    r = rv[jj, pl.ds(s, _L)]
                    g = gv[jj, pl.ds(s, _L)]
                    b_ = bv[jj, pl.ds(s, _L)]
                    rs = r * inv
                    gs = g * inv
                    bs = b_ * inv
                    ri = rs.astype(jnp.int32)
                    gi = gs.astype(jnp.int32)
                    bi = bs.astype(jnp.int32)
                    rd = rs - ri.astype(jnp.float32)
                    gd = gs - gi.astype(jnp.float32)
                    bd = bs - bi.astype(jnp.float32)
                    base_id = ri + gi * _DIM + bi * _DIM2

                    def unpk(w):
                        lo = plsc.bitcast(w << 16, jnp.float32)
                        hi = plsc.bitcast(w & jnp.int32(-65536), jnp.float32)
                        return lo, hi

                    res = []
                    for c in range(3):
                        bc = base_id + c * _TBL
                        v000, v100 = unpk(plsc.load_gather(lut_v, [bc]))
                        v010, v110 = unpk(plsc.load_gather(lut_v, [bc + _DIM]))
                        v001, v101 = unpk(plsc.load_gather(lut_v, [bc + _DIM2]))
                        v011, v111 = unpk(
                            plsc.load_gather(lut_v, [bc + (_DIM2 + _DIM)]))
                        v00 = v000 + rd * (v100 - v000)
                        v10 = v010 + rd * (v110 - v010)
                        v01 = v001 + rd * (v101 - v001)
                        v11 = v011 + rd * (v111 - v011)
                        v0 = v00 + gd * (v10 - v00)
                        v1 = v01 + gd * (v11 - v01)
                        res.append(v0 + bd * (v1 - v0))
                    ov0[jj, pl.ds(s, _L)] = res[0]
                    ov1[jj, pl.ds(s, _L)] = res[1]
                    ov2[jj, pl.ds(s, _L)] = res[2]

            for cp in out_copies(ci, p):
                cp.start()

    for cp in out_copies(_NCHUNK - 2, 0):
        cp.wait()
    for cp in out_copies(_NCHUNK - 1, 1):
        cp.wait()


def _pack_pairs(lut):
    t = lut.reshape(3, _TBL)
    tn = jnp.concatenate([t[:, 1:], t[:, -1:]], axis=1)
    lo = lax.bitcast_convert_type(t.astype(jnp.bfloat16), jnp.uint16)
    hi = lax.bitcast_convert_type(tn.astype(jnp.bfloat16), jnp.uint16)
    packed = lo.astype(jnp.uint32) | (hi.astype(jnp.uint32) << 16)
    return lax.bitcast_convert_type(packed, jnp.int32).reshape(3 * _TBL)


def kernel(lut, x):
    return _lut3d_sc(_pack_pairs(lut), x)
